# baseline (speedup 1.0000x reference)
import jax
import jax.numpy as jnp
from jax.experimental import pallas as pl
from jax.experimental.pallas import tpu as pltpu

K = 17
N = 50000
IC = 32
OC = 32
B = 1024
G = (N + K - 1 + B - 1) // B
NSLOT = 3


def _copy(in_refs, xbuf, sems, slot, kind, bi):
    copies = []
    for k in range(K):
        if kind == 'first':
            src = in_refs[k].at[pl.ds(0, B - k)]
            dst = xbuf.at[slot, k, pl.ds(k, B - k), :]
        elif kind == 'last':
            s = (G - 1) * B - k
            L = N - s
            src = in_refs[k].at[pl.ds(s, L)]
            dst = xbuf.at[slot, k, pl.ds(0, L), :]
        else:
            s = bi * B - k
            src = in_refs[k].at[pl.ds(s, B)]
            dst = xbuf.at[slot, k]
        copies.append(pltpu.make_async_copy(src, dst, sems.at[slot, k]))
    return copies


def _body(w_ref, b_ref, *refs):
    in_refs = refs[:K]
    out_ref = refs[K]
    xbuf = refs[K + 1]
    sems = refs[K + 2]

    i = pl.program_id(0)
    slot = jax.lax.rem(i, NSLOT)
    nslot = jax.lax.rem(i + 1, NSLOT)

    @pl.when(i == 0)
    def _prologue():
        for c in _copy(in_refs, xbuf, sems, 0, 'first', 0):
            c.start()

    @pl.when(i < G - 2)
    def _prefetch_interior():
        for c in _copy(in_refs, xbuf, sems, nslot, 'interior', i + 1):
            c.start()

    @pl.when(i == G - 2)
    def _prefetch_last():
        for c in _copy(in_refs, xbuf, sems, nslot, 'last', G - 1):
            c.start()

    @pl.when(i == 0)
    def _wait_first():
        for c in _copy(in_refs, xbuf, sems, slot, 'first', 0):
            c.wait()

    @pl.when(jnp.logical_and(i > 0, i < G - 1))
    def _wait_interior():
        for c in _copy(in_refs, xbuf, sems, slot, 'interior', i):
            c.wait()

    @pl.when(i == G - 1)
    def _wait_last():
        for c in _copy(in_refs, xbuf, sems, slot, 'last', G - 1):
            c.wait()

    def matsum(parts):
        acc = None
        for k in range(K):
            p = jax.lax.dot_general(
                parts[k], w_ref[k], (((1,), (1,)), ((), ())),
                preferred_element_type=jnp.float32)
            acc = p if acc is None else acc + p
        return acc

    @pl.when(jnp.logical_and(i > 0, i < G - 1))
    def _fast():
        acc = matsum([xbuf[slot, k] for k in range(K)])
        out_ref[...] = acc + jnp.sum(b_ref[...], axis=0, keepdims=True)

    @pl.when(jnp.logical_or(i == 0, i == G - 1))
    def _edge():
        m1 = jax.lax.broadcasted_iota(jnp.int32, (B, 1), 0) + i * B
        masked = []
        mask_cols = []
        for k in range(K):
            valid = jnp.logical_and(m1 >= k, m1 <= (N - 1) + k)
            masked.append(jnp.where(valid, xbuf[slot, k], 0.0))
            mask_cols.append(valid.astype(jnp.float32))
        acc = matsum(masked)
        maskf = jnp.concatenate(mask_cols, axis=1)
        out_ref[...] = acc + jax.lax.dot_general(
            maskf, b_ref[...], (((1,), (0,)), ((), ())),
            preferred_element_type=jnp.float32)


def kernel(weights, bias, input_0, input_1, input_2, input_3, input_4,
           input_5, input_6, input_7, input_8, input_9, input_10, input_11,
           input_12, input_13, input_14, input_15, input_16):
    ins = (input_0, input_1, input_2, input_3, input_4, input_5, input_6,
           input_7, input_8, input_9, input_10, input_11, input_12, input_13,
           input_14, input_15, input_16)
    n_out = N + K - 1
    return pl.pallas_call(
        _body,
        grid=(G,),
        in_specs=[
            pl.BlockSpec((K, OC, IC), lambda i: (0, 0, 0)),
            pl.BlockSpec((K, OC), lambda i: (0, 0)),
        ] + [pl.BlockSpec(memory_space=pl.ANY)] * K,
        out_specs=pl.BlockSpec((B, OC), lambda i: (i, 0)),
        out_shape=jax.ShapeDtypeStruct((n_out, OC), jnp.float32),
        scratch_shapes=[
            pltpu.VMEM((NSLOT, K, B, IC), jnp.float32),
            pltpu.SemaphoreType.DMA((NSLOT, K)),
        ],
        compiler_params=pltpu.CompilerParams(
            dimension_semantics=("arbitrary",)),
    )(weights, bias, *ins)


# speedup vs baseline: 8.1261x; 8.1261x over previous
import jax
import jax.numpy as jnp
from jax.experimental import pallas as pl
from jax.experimental.pallas import tpu as pltpu

K = 17
N = 50000
IC = 32
OC = 32
L = 4096
NO = N + K - 1
G = (NO + L - 1) // L
HALO = 128


def _body(w_ref, b_ref, *refs):
    cur = refs[:K]
    prev = refs[K:2 * K]
    out_ref = refs[2 * K]

    i = pl.program_id(0)
    num = pl.num_programs(0)

    def compute(masked):
        acc = None
        for k in range(K):
            z = jnp.concatenate(
                [prev[k][:, HALO - (K - 1):], cur[k][...]], axis=1)
            sh = jax.lax.slice(z, (0, K - 1 - k), (IC, K - 1 - k + L))
            if masked:
                mcol = jax.lax.broadcasted_iota(jnp.int32, (IC, L), 1) + i * L
                valid = jnp.logical_and(mcol >= k, mcol <= (N - 1) + k)
                sh = jnp.where(valid, sh, 0.0)
            p = jax.lax.dot_general(
                w_ref[k], sh, (((1,), (0,)), ((), ())),
                preferred_element_type=jnp.float32)
            acc = p if acc is None else acc + p
        if masked:
            mcol = jax.lax.broadcasted_iota(jnp.int32, (IC, L), 1) + i * L
            for k in range(K):
                vk = jnp.logical_and(mcol >= k, mcol <= (N - 1) + k)
                acc = acc + jnp.where(vk, b_ref[k][:, None], 0.0)
        else:
            acc = acc + jnp.sum(b_ref[...], axis=0)[:, None]
        out_ref[...] = acc

    @pl.when(jnp.logical_and(i > 0, i < num - 1))
    def _fast():
        compute(False)

    @pl.when(jnp.logical_or(i == 0, i == num - 1))
    def _edge():
        compute(True)


def kernel(weights, bias, input_0, input_1, input_2, input_3, input_4,
           input_5, input_6, input_7, input_8, input_9, input_10, input_11,
           input_12, input_13, input_14, input_15, input_16):
    ins = (input_0, input_1, input_2, input_3, input_4, input_5, input_6,
           input_7, input_8, input_9, input_10, input_11, input_12, input_13,
           input_14, input_15, input_16)
    xts = tuple(jnp.transpose(x) for x in ins)

    cur_spec = pl.BlockSpec((IC, L), lambda i: (0, i))
    prev_spec = pl.BlockSpec(
        (IC, HALO), lambda i: (0, jnp.maximum(i * (L // HALO) - 1, 0)))
    outt = pl.pallas_call(
        _body,
        grid=(G,),
        in_specs=[
            pl.BlockSpec((K, OC, IC), lambda i: (0, 0, 0)),
            pl.BlockSpec((K, OC), lambda i: (0, 0)),
        ] + [cur_spec] * K + [prev_spec] * K,
        out_specs=pl.BlockSpec((OC, L), lambda i: (0, i)),
        out_shape=jax.ShapeDtypeStruct((OC, NO), jnp.float32),
        compiler_params=pltpu.CompilerParams(
            dimension_semantics=("arbitrary",)),
    )(weights, bias, *xts, *xts)
    return jnp.transpose(outt)
